# baseline (speedup 1.0000x reference)
import jax
import jax.numpy as jnp
from jax.experimental import pallas as pl
from jax.experimental.pallas import tpu as pltpu

HIDDEN = 1024
FFN = 2048
E = 16
TOPK = 2
FB = 1024
NF = FFN // FB
NSTEP = E * NF
NSLOT = 3


def _moe_kernel(x_ref, tw_ref, te_ref, up_hbm, gate_hbm, down_hbm, out_ref,
                up_buf, gate_buf, down_buf, up_sem, gate_sem, down_sem):
    xf = x_ref[...]

    def start(s):
        slot = s % NSLOT
        e, f = divmod(s, NF)
        pltpu.make_async_copy(
            up_hbm.at[e, pl.ds(f * FB, FB), :], up_buf.at[slot],
            up_sem.at[slot]).start()
        pltpu.make_async_copy(
            gate_hbm.at[e, pl.ds(f * FB, FB), :], gate_buf.at[slot],
            gate_sem.at[slot]).start()
        pltpu.make_async_copy(
            down_hbm.at[e, :, pl.ds(f * FB, FB)], down_buf.at[slot],
            down_sem.at[slot]).start()

    def wait(s):
        slot = s % NSLOT
        e, f = divmod(s, NF)
        pltpu.make_async_copy(
            up_hbm.at[e, pl.ds(f * FB, FB), :], up_buf.at[slot],
            up_sem.at[slot]).wait()
        pltpu.make_async_copy(
            gate_hbm.at[e, pl.ds(f * FB, FB), :], gate_buf.at[slot],
            gate_sem.at[slot]).wait()
        pltpu.make_async_copy(
            down_hbm.at[e, :, pl.ds(f * FB, FB)], down_buf.at[slot],
            down_sem.at[slot]).wait()

    for s in range(NSLOT):
        start(s)

    for s in range(NSTEP):
        slot = s % NSLOT
        e = s // NF
        wait(s)

        up = up_buf[slot]
        gate = gate_buf[slot]
        down = down_buf[slot]

        x1 = jax.lax.dot_general(xf, up, (((1,), (1,)), ((), ())),
                                 preferred_element_type=jnp.float32)
        x2 = jax.lax.dot_general(xf, gate, (((1,), (1,)), ((), ())),
                                 preferred_element_type=jnp.float32)
        h = x1 * jax.nn.sigmoid(x1) * x2
        partial = jax.lax.dot_general(h, down, (((1,), (1,)), ((), ())),
                                      preferred_element_type=jnp.float32)

        mask = te_ref[...] == e
        w = jnp.sum(jnp.where(mask, tw_ref[...], 0.0), axis=1, keepdims=True)
        contrib = partial * w

        if s == 0:
            out_ref[...] = contrib
        else:
            out_ref[...] += contrib

        if s + NSLOT < NSTEP:
            start(s + NSLOT)


def kernel(x, weights, top_weights, top_experts, up_w, gate_w, down_w):
    bsz, q_len, hidden = x.shape
    tok = bsz * q_len
    xf = x.reshape(tok, hidden)
    te = top_experts.astype(jnp.int32)

    out = pl.pallas_call(
        _moe_kernel,
        in_specs=[
            pl.BlockSpec((tok, hidden), lambda: (0, 0)),
            pl.BlockSpec((tok, TOPK), lambda: (0, 0)),
            pl.BlockSpec((tok, TOPK), lambda: (0, 0)),
            pl.BlockSpec(memory_space=pltpu.MemorySpace.HBM),
            pl.BlockSpec(memory_space=pltpu.MemorySpace.HBM),
            pl.BlockSpec(memory_space=pltpu.MemorySpace.HBM),
        ],
        out_specs=pl.BlockSpec((tok, hidden), lambda: (0, 0)),
        out_shape=jax.ShapeDtypeStruct((tok, hidden), jnp.float32),
        scratch_shapes=[
            pltpu.VMEM((NSLOT, FB, hidden), jnp.float32),
            pltpu.VMEM((NSLOT, FB, hidden), jnp.float32),
            pltpu.VMEM((NSLOT, hidden, FB), jnp.float32),
            pltpu.SemaphoreType.DMA((NSLOT,)),
            pltpu.SemaphoreType.DMA((NSLOT,)),
            pltpu.SemaphoreType.DMA((NSLOT,)),
        ],
    )(xf, top_weights, te, up_w, gate_w, down_w)

    return out.reshape(bsz, q_len, hidden)


# speedup vs baseline: 1.0046x; 1.0046x over previous
import jax
import jax.numpy as jnp
from jax.experimental import pallas as pl

HIDDEN = 1024
FFN = 2048
E = 16
TOPK = 2
FB = 1024
NF = FFN // FB


def _moe_kernel(x_ref, tw_ref, te_ref, up_ref, gate_ref, down_ref, out_ref):
    e = pl.program_id(0)
    f = pl.program_id(1)

    xf = x_ref[...]
    up = up_ref[0]
    gate = gate_ref[0]
    down = down_ref[0]

    x1 = jax.lax.dot_general(xf, up, (((1,), (1,)), ((), ())),
                             preferred_element_type=jnp.float32)
    x2 = jax.lax.dot_general(xf, gate, (((1,), (1,)), ((), ())),
                             preferred_element_type=jnp.float32)
    h = x1 * jax.nn.sigmoid(x1) * x2
    partial = jax.lax.dot_general(h, down, (((1,), (1,)), ((), ())),
                                  preferred_element_type=jnp.float32)

    mask = te_ref[...] == e
    w = jnp.sum(jnp.where(mask, tw_ref[...], 0.0), axis=1, keepdims=True)
    contrib = partial * w

    first = (e == 0) & (f == 0)

    @pl.when(first)
    def _():
        out_ref[...] = contrib

    @pl.when(jnp.logical_not(first))
    def _():
        out_ref[...] += contrib


def kernel(x, weights, top_weights, top_experts, up_w, gate_w, down_w):
    bsz, q_len, hidden = x.shape
    tok = bsz * q_len
    xf = x.reshape(tok, hidden)
    te = top_experts.astype(jnp.int32)

    out = pl.pallas_call(
        _moe_kernel,
        grid=(E, NF),
        in_specs=[
            pl.BlockSpec((tok, hidden), lambda e, f: (0, 0)),
            pl.BlockSpec((tok, TOPK), lambda e, f: (0, 0)),
            pl.BlockSpec((tok, TOPK), lambda e, f: (0, 0)),
            pl.BlockSpec((1, FB, hidden), lambda e, f: (e, f, 0)),
            pl.BlockSpec((1, FB, hidden), lambda e, f: (e, f, 0)),
            pl.BlockSpec((1, hidden, FB), lambda e, f: (e, 0, f)),
        ],
        out_specs=pl.BlockSpec((tok, hidden), lambda e, f: (0, 0)),
        out_shape=jax.ShapeDtypeStruct((tok, hidden), jnp.float32),
    )(xf, top_weights, te, up_w, gate_w, down_w)

    return out.reshape(bsz, q_len, hidden)
